# baseline (speedup 1.0000x reference)
import jax
import jax.numpy as jnp
from jax.experimental import pallas as pl

N_TOK = 8192
DIM = 4096
BM = 512
BN = 2048


def _mm_kernel(x_ref, w_ref, b_ref, o_ref):
    acc = jnp.dot(x_ref[...], w_ref[...], preferred_element_type=jnp.float32)
    o_ref[...] = acc + b_ref[...]


def kernel(x, weight, bias):
    xb = x.astype(jnp.bfloat16)
    wb = weight.astype(jnp.bfloat16)
    b2 = bias.reshape(1, DIM)
    grid = (DIM // BN, N_TOK // BM)
    return pl.pallas_call(
        _mm_kernel,
        grid=grid,
        in_specs=[
            pl.BlockSpec((BM, DIM), lambda n, m: (m, 0)),
            pl.BlockSpec((DIM, BN), lambda n, m: (0, n)),
            pl.BlockSpec((1, BN), lambda n, m: (0, n)),
        ],
        out_specs=pl.BlockSpec((BM, BN), lambda n, m: (m, n)),
        out_shape=jax.ShapeDtypeStruct((N_TOK, DIM), jnp.float32),
    )(xb, wb, b2)
